# Initial kernel scaffold; baseline (speedup 1.0000x reference)
import jax
import jax.numpy as jnp
from jax.experimental import pallas as pl


def kernel(params, edges):
    raise NotImplementedError("write your pallas kernel here")


# speedup vs baseline: 1.0821x; 1.0821x over previous
import functools

import jax
import jax.numpy as jnp
from jax import lax
from jax.experimental import pallas as pl
from jax.experimental.pallas import tpu as pltpu
from jax.experimental.pallas import tpu_sc as plsc

_NODE = {"drug": 20000, "disease": 20000, "gene": 60000}
_ETS = [("drug", "targets", "gene"), ("gene", "assoc", "disease"),
        ("gene", "rev_targets", "drug"), ("disease", "rev_assoc", "gene")]
_D = 128
_NC, _NS, _L = 2, 16, 16
_B = 128


def _ekey(s, r, d):
    return s + "_" + r + "_" + d


def _ranges(n_dst):
    n_ranges = 6 if n_dst > 32000 else 2
    rpt = -(-n_dst // (n_ranges * _NS * 8)) * 8
    return n_ranges, _NS * rpt, rpt


def _remap(dst_ref, out_ref, lo, rng):
    for i in range(_B // _L):
        v = dst_ref[pl.ds(i * _L, _L)]
        lv = v - lo
        ok = (lv >= 0) & (lv < rng)
        out_ref[pl.ds(i * _L, _L)] = jnp.where(ok, lv, rng)


@functools.lru_cache(maxsize=None)
def _agg_kernel(n_src, n_dst, nb):
    n_ranges, rng, rpt = _ranges(n_dst)
    mesh = plsc.VectorSubcoreMesh(core_axis_name="c", subcore_axis_name="s")

    def body(x_hbm, srcb, dstb, zeros_hbm, out_hbm,
             acc, idxs, idxd, idxd2, rows, sem):
        c = lax.axis_index("c")
        s = lax.axis_index("s")

        def one_pass(q):
            lo = q * rng
            pltpu.sync_copy(zeros_hbm.at[pl.ds(0, rpt)],
                            acc.at[pl.ds(s * rpt, rpt)])
            plsc.subcore_barrier()

            def blk(j, carry):
                pltpu.sync_copy(srcb.at[s, j], idxs)
                pltpu.sync_copy(dstb.at[s, j], idxd)
                _remap(idxd, idxd2, lo, rng)
                pltpu.async_copy(x_hbm.at[idxs], rows, sem).wait()
                pltpu.sync_copy(rows, acc.at[idxd2], add=True)
                return carry

            lax.fori_loop(0, nb, blk, 0)
            plsc.subcore_barrier()
            pltpu.sync_copy(acc.at[pl.ds(s * rpt, rpt)],
                            out_hbm.at[pl.ds(lo + s * rpt, rpt)])
            plsc.subcore_barrier()

        for half in range(_NC):
            @pl.when(c == half)
            def _():
                for q in range(half, n_ranges, _NC):
                    one_pass(q)

    return pl.kernel(
        body,
        out_type=jax.ShapeDtypeStruct((n_ranges * rng, _D), jnp.float32),
        mesh=mesh,
        scratch_types=[
            pltpu.VMEM_SHARED((rng + 8, _D), jnp.float32),
            pltpu.VMEM((_B,), jnp.int32),
            pltpu.VMEM((_B,), jnp.int32),
            pltpu.VMEM((_B,), jnp.int32),
            pltpu.VMEM((_B, _D), jnp.float32),
            pltpu.SemaphoreType.DMA,
        ])


@functools.lru_cache(maxsize=None)
def _cnt_kernel(n_dst, nb):
    n_ranges, rng, rpt = _ranges(n_dst)
    mesh = plsc.VectorSubcoreMesh(core_axis_name="c", subcore_axis_name="s")

    def body(dstb, zeros_hbm, ones_hbm, out_hbm,
             acc, idxd, idxd2, ones_v, sem):
        c = lax.axis_index("c")
        s = lax.axis_index("s")
        pltpu.sync_copy(ones_hbm, ones_v)

        def one_pass(q):
            lo = q * rng
            pltpu.sync_copy(zeros_hbm.at[pl.ds(0, rpt)],
                            acc.at[pl.ds(s * rpt, rpt)])
            plsc.subcore_barrier()

            def blk(j, carry):
                pltpu.sync_copy(dstb.at[s, j], idxd)
                _remap(idxd, idxd2, lo, rng)
                pltpu.sync_copy(ones_v, acc.at[idxd2], add=True)
                return carry

            lax.fori_loop(0, nb, blk, 0)
            plsc.subcore_barrier()
            pltpu.sync_copy(acc.at[pl.ds(s * rpt, rpt)],
                            out_hbm.at[pl.ds(lo + s * rpt, rpt)])
            plsc.subcore_barrier()

        for half in range(_NC):
            @pl.when(c == half)
            def _():
                for q in range(half, n_ranges, _NC):
                    one_pass(q)

    return pl.kernel(
        body,
        out_type=jax.ShapeDtypeStruct((n_ranges * rng, _D), jnp.float32),
        mesh=mesh,
        scratch_types=[
            pltpu.VMEM_SHARED((rng + 8, _D), jnp.float32),
            pltpu.VMEM((_B,), jnp.int32),
            pltpu.VMEM((_B,), jnp.int32),
            pltpu.VMEM((_B, _D), jnp.float32),
            pltpu.SemaphoreType.DMA,
        ])


_R = 1000


@functools.lru_cache(maxsize=None)
def _combine_kernel(n, n_et):
    def body(*refs):
        aggs = refs[0:n_et]
        recips = refs[n_et:2 * n_et]
        x_ref = refs[2 * n_et]
        wls = refs[2 * n_et + 1:3 * n_et + 1]
        wr_ref = refs[3 * n_et + 1]
        b_ref = refs[3 * n_et + 2]
        out_ref = refs[3 * n_et + 3]
        acc = jnp.dot(x_ref[...], wr_ref[...],
                      preferred_element_type=jnp.float32) + b_ref[...]
        for a, r, w in zip(aggs, recips, wls):
            acc = acc + jnp.dot(a[...] * r[...], w[...],
                                preferred_element_type=jnp.float32)
        out_ref[...] = jnp.maximum(acc, 0.0)

    row_spec = pl.BlockSpec((_R, _D), lambda i: (i, 0))
    one_spec = pl.BlockSpec((_R, 1), lambda i: (i, 0))
    w_spec = pl.BlockSpec((_D, _D), lambda i: (0, 0))
    b_spec = pl.BlockSpec((1, _D), lambda i: (0, 0))
    in_specs = ([row_spec] * n_et + [one_spec] * n_et + [row_spec]
                + [w_spec] * n_et + [w_spec, b_spec])
    return pl.pallas_call(
        body,
        grid=(n // _R,),
        in_specs=in_specs,
        out_specs=row_spec,
        out_shape=jax.ShapeDtypeStruct((n, _D), jnp.float32),
    )


def _pad_blocks(a, nb, fill):
    total = _NS * nb * _B
    a = jnp.concatenate(
        [a, jnp.full((total - a.shape[0],), fill, jnp.int32)])
    return a.reshape(_NS, nb, _B)


def kernel(params, edges):
    blocks = {}
    for (s, r, d) in _ETS:
        k = _ekey(s, r, d)
        e = edges[k]
        E = e.shape[1]
        n_dst = _NODE[d]
        _, rng, _ = _ranges(n_dst)
        nb = -(-E // (_NS * _B))
        srcb = _pad_blocks(e[0], nb, 0)
        dstb = _pad_blocks(e[1], nb, n_dst)
        blocks[k] = (srcb, dstb, nb)

    x = {nt: params["emb"][nt] for nt in _NODE}
    cnt_recip = {}
    for (s, r, d) in _ETS:
        k = _ekey(s, r, d)
        srcb, dstb, nb = blocks[k]
        n_dst = _NODE[d]
        _, _, rpt = _ranges(n_dst)
        zeros = jnp.zeros((rpt, _D), jnp.float32)
        ones = jnp.ones((_B, _D), jnp.float32)
        cnt = _cnt_kernel(n_dst, nb)(dstb, zeros, ones)[:n_dst, 0]
        cnt_recip[k] = (1.0 / jnp.maximum(cnt, 1.0)).reshape(n_dst, 1)

    for l in range(2):
        lp = params["l" + str(l)]
        aggs = {}
        for (s, r, d) in _ETS:
            k = _ekey(s, r, d)
            srcb, dstb, nb = blocks[k]
            n_dst = _NODE[d]
            _, _, rpt = _ranges(n_dst)
            zeros = jnp.zeros((rpt, _D), jnp.float32)
            aggs[k] = _agg_kernel(_NODE[s], n_dst, nb)(
                x[s], srcb, dstb, zeros)
        new_x = {}
        for nt in _NODE:
            ets = [(s, r, d) for (s, r, d) in _ETS if d == nt]
            n_et = len(ets)
            ks = [_ekey(*et) for et in ets]
            wr = sum(lp[k]["W_r"] for k in ks) / n_et
            bb = (sum(lp[k]["b_l"] for k in ks) / n_et).reshape(1, _D)
            args = ([aggs[k] for k in ks]
                    + [cnt_recip[k] for k in ks]
                    + [x[nt]] + [lp[k]["W_l"] / n_et for k in ks]
                    + [wr, bb])
            new_x[nt] = _combine_kernel(_NODE[nt], n_et)(*args)
        x = new_x
    return (x["drug"], x["disease"], x["gene"])
